# baseline (speedup 1.0000x reference)
import functools

import jax
import jax.numpy as jnp
from jax import lax
from jax.experimental import pallas as pl
from jax.experimental.pallas import tpu as pltpu
from jax.experimental.pallas import tpu_sc as plsc

N = 10000
D = 128
E = 320000
NC = 2
NS = 16
NW = NC * NS
CH = 128
NCHUNK = 80
EP = NCHUNK * CH
NBUF = 2
NP = 10240
RPT = NP // NS
RCH = RPT // CH


def _fill_const_2d(ref, nrows, ncols, value):
    vec = jnp.full((16,), value, dtype=jnp.float32)

    def row(r, _):
        def col(c, _):
            ref[r, pl.ds(c * 16, 16)] = vec
            return 0
        return lax.fori_loop(0, ncols // 16, col, 0)

    lax.fori_loop(0, nrows, row, 0)


@functools.lru_cache(maxsize=None)
def _make_sc_conv(with_deg):
    mesh = plsc.VectorSubcoreMesh(core_axis_name="c", subcore_axis_name="s",
                                  num_cores=NC, num_subcores=NS)
    out_type = [jax.ShapeDtypeStruct((NC * NP, D), jnp.float32)]
    if with_deg:
        out_type.append(jax.ShapeDtypeStruct((NC * NP, D), jnp.float32))
    scratch = (
        [pltpu.VMEM((NCHUNK, CH), jnp.int32)]
        + [pltpu.VMEM((CH,), jnp.int32) for _ in range(NBUF)]
        + [pltpu.VMEM((CH, D), jnp.float32) for _ in range(NBUF)]
        + [pltpu.SemaphoreType.DMA for _ in range(NBUF)]
        + [pltpu.VMEM_SHARED((NP, D), jnp.float32)]
    )

    def body(h_hbm, src_hbm, dst_hbm, *rest):
        if with_deg:
            (p_out, degp_out, dst_v, *rs, acc_sh) = rest
        else:
            (p_out, dst_v, *rs, acc_sh) = rest
        srcs = rs[:NBUF]
        rows = rs[NBUF:2 * NBUF]
        sems = rs[2 * NBUF:]
        cid = lax.axis_index("c")
        sid = lax.axis_index("s")
        wid = sid * NC + cid
        r0 = sid * RPT
        obase = cid * NP

        def zero_acc():
            def zstep(i, _):
                pltpu.sync_copy(rows[0], acc_sh.at[pl.ds(r0 + i * CH, CH)])
                return 0

            lax.fori_loop(0, RCH, zstep, 0)

        def dump_acc(out):
            def dstep(i, _):
                off = r0 + i * CH
                pltpu.sync_copy(acc_sh.at[pl.ds(off, CH)], rows[0])
                pltpu.sync_copy(rows[0], out.at[pl.ds(obase + off, CH)])
                return 0

            lax.fori_loop(0, RCH, dstep, 0)

        pltpu.sync_copy(dst_hbm.at[wid], dst_v)

        _fill_const_2d(rows[0], CH, D, 0.0)
        zero_acc()
        plsc.subcore_barrier()

        def estep(k, _):
            pltpu.sync_copy(src_hbm.at[wid, k], srcs[0])
            pltpu.async_copy(h_hbm.at[srcs[0]], rows[0], sems[0]).wait()
            pltpu.sync_copy(rows[0], acc_sh.at[dst_v.at[k]], add=True)
            return 0

        lax.fori_loop(0, NCHUNK, estep, 0)
        plsc.subcore_barrier()

        dump_acc(p_out)

        if with_deg:
            _fill_const_2d(rows[0], CH, D, 0.0)
            zero_acc()
            plsc.subcore_barrier()
            _fill_const_2d(rows[1], CH, D, 1.0)

            def gstep(k, _):
                pltpu.sync_copy(rows[1], acc_sh.at[dst_v.at[k]], add=True)
                return 0

            lax.fori_loop(0, NCHUNK, gstep, 0)
            plsc.subcore_barrier()
            dump_acc(degp_out)

    return pl.kernel(body, out_type=out_type, mesh=mesh,
                     scratch_types=scratch)


_R = 2000


def _proj_kernel(x_ref, w_ref, o_ref):
    o_ref[...] = jnp.maximum(
        jnp.dot(x_ref[...], w_ref[...], preferred_element_type=jnp.float32),
        0.0)


def _proj_relu(x, w):
    return pl.pallas_call(
        _proj_kernel,
        grid=(N // _R,),
        in_specs=[pl.BlockSpec((_R, D), lambda i: (i, 0)),
                  pl.BlockSpec((D, D), lambda i: (0, 0))],
        out_specs=pl.BlockSpec((_R, D), lambda i: (i, 0)),
        out_shape=jax.ShapeDtypeStruct((N, D), jnp.float32),
    )(x, w)


def _mid_kernel(coeff_ref, p_ref, degp_ref, basis_ref, h_ref, inv_ref):
    w = coeff_ref[0, 0] * basis_ref[0] + coeff_ref[0, 1] * basis_ref[1]
    agg = p_ref[0] + p_ref[1]
    deg = degp_ref[0] + degp_ref[1]
    inv = 1.0 / jnp.maximum(deg, 1.0)
    inv_ref[...] = inv
    m = jnp.dot(agg, w, preferred_element_type=jnp.float32) * inv
    h_ref[...] = jnp.maximum(m, 0.0)


def _mid(p, degp, basis, coeff_row):
    return pl.pallas_call(
        _mid_kernel,
        grid=(N // _R,),
        in_specs=[pl.BlockSpec(memory_space=pltpu.SMEM),
                  pl.BlockSpec((NC, _R, D), lambda i: (0, i, 0)),
                  pl.BlockSpec((NC, _R, D), lambda i: (0, i, 0)),
                  pl.BlockSpec((2, D, D), lambda i: (0, 0, 0))],
        out_specs=[pl.BlockSpec((_R, D), lambda i: (i, 0)),
                   pl.BlockSpec((_R, D), lambda i: (i, 0))],
        out_shape=[jax.ShapeDtypeStruct((N, D), jnp.float32),
                   jax.ShapeDtypeStruct((N, D), jnp.float32)],
    )(coeff_row, p, degp, basis)


def _final_kernel(coeff_ref, p_ref, inv_ref, basis_ref, wdec_ref, bdec_ref,
                  feat_ref, h_ref, loss_ref):
    w = coeff_ref[0, 0] * basis_ref[0] + coeff_ref[0, 1] * basis_ref[1]
    agg = p_ref[0] + p_ref[1]
    h = jnp.dot(agg, w, preferred_element_type=jnp.float32) * inv_ref[...]
    h_ref[...] = h
    rec = (jnp.dot(h, wdec_ref[...], preferred_element_type=jnp.float32)
           + bdec_ref[...])
    dlt = rec - feat_ref[...]
    part = jnp.sum(dlt * dlt) / (N * D)

    @pl.when(pl.program_id(0) == 0)
    def _():
        loss_ref[0, 0] = 0.0

    loss_ref[0, 0] += part


def _final(p, inv, basis, coeff_row, wdec, bdec, feat):
    return pl.pallas_call(
        _final_kernel,
        grid=(N // _R,),
        in_specs=[pl.BlockSpec(memory_space=pltpu.SMEM),
                  pl.BlockSpec((NC, _R, D), lambda i: (0, i, 0)),
                  pl.BlockSpec((_R, D), lambda i: (i, 0)),
                  pl.BlockSpec((2, D, D), lambda i: (0, 0, 0)),
                  pl.BlockSpec((D, D), lambda i: (0, 0)),
                  pl.BlockSpec((1, D), lambda i: (0, 0)),
                  pl.BlockSpec((_R, D), lambda i: (i, 0))],
        out_specs=[pl.BlockSpec((_R, D), lambda i: (i, 0)),
                   pl.BlockSpec(memory_space=pltpu.SMEM)],
        out_shape=[jax.ShapeDtypeStruct((N, D), jnp.float32),
                   jax.ShapeDtypeStruct((1, 1), jnp.float32)],
    )(coeff_row, p, inv, basis, wdec, bdec.reshape(1, D), feat)


def kernel(feat_user, feat_item, edge_u2i, edge_i2u,
           W_in_user, W_in_item, basis1, coeff1, basis2, coeff2,
           W_dec_user, b_dec_user, W_dec_item, b_dec_item):
    def pad_edges(idx, fill):
        idx = idx.astype(jnp.int32)
        pad = jnp.full((NW * EP - E,), fill, jnp.int32)
        return jnp.concatenate([idx, pad]).reshape(NW, NCHUNK, CH)

    su = pad_edges(edge_u2i[0], 0)
    du = pad_edges(edge_u2i[1], N)
    si = pad_edges(edge_i2u[0], 0)
    di = pad_edges(edge_i2u[1], N)

    h_u = _proj_relu(feat_user, W_in_user)
    h_i = _proj_relu(feat_item, W_in_item)

    _sc_conv_deg = _make_sc_conv(True)
    _sc_conv = _make_sc_conv(False)

    p_item, degp_item = _sc_conv_deg(h_u, su, du)
    p_user, degp_user = _sc_conv_deg(h_i, si, di)
    p_item = p_item.reshape(NC, NP, D)
    p_user = p_user.reshape(NC, NP, D)
    degp_item = degp_item.reshape(NC, NP, D)
    degp_user = degp_user.reshape(NC, NP, D)
    h_i1, inv_i = _mid(p_item, degp_item, basis1, coeff1[0:1])
    h_u1, inv_u = _mid(p_user, degp_user, basis1, coeff1[1:2])

    (p_item2,) = _sc_conv(h_u1, su, du)
    (p_user2,) = _sc_conv(h_i1, si, di)
    p_item2 = p_item2.reshape(NC, NP, D)
    p_user2 = p_user2.reshape(NC, NP, D)
    h_i2, loss_i = _final(p_item2, inv_i, basis2, coeff2[0:1],
                          W_dec_item, b_dec_item, feat_item)
    h_u2, loss_u = _final(p_user2, inv_u, basis2, coeff2[1:2],
                          W_dec_user, b_dec_user, feat_user)

    loss = loss_u[0, 0] + loss_i[0, 0]
    return (loss, h_u2, h_i2)


# speedup vs baseline: 1.9504x; 1.9504x over previous
import functools

import jax
import jax.numpy as jnp
from jax import lax
from jax.experimental import pallas as pl
from jax.experimental.pallas import tpu as pltpu
from jax.experimental.pallas import tpu_sc as plsc

N = 10000
D = 128
E = 320000
NC = 2
NS = 16
NW = NC * NS
CH = 80
NCHUNK = 125
EP = NCHUNK * CH
NBUF = 2
NP = 10240
RPT = NP // NS
RCH = RPT // CH


def _fill_const_2d(ref, nrows, ncols, value):
    vec = jnp.full((16,), value, dtype=jnp.float32)

    def row(r, _):
        def col(c, _):
            ref[r, pl.ds(c * 16, 16)] = vec
            return 0
        return lax.fori_loop(0, ncols // 16, col, 0)

    lax.fori_loop(0, nrows, row, 0)


@functools.lru_cache(maxsize=None)
def _make_sc_conv(with_deg):
    mesh = plsc.VectorSubcoreMesh(core_axis_name="c", subcore_axis_name="s",
                                  num_cores=NC, num_subcores=NS)
    out_type = [jax.ShapeDtypeStruct((NC * NP, D), jnp.float32)]
    if with_deg:
        out_type.append(jax.ShapeDtypeStruct((NC * NP, D), jnp.float32))
    scratch = (
        [pltpu.VMEM((NCHUNK, CH), jnp.int32)]
        + [pltpu.VMEM((CH,), jnp.int32) for _ in range(NBUF)]
        + [pltpu.VMEM((CH, D), jnp.float32) for _ in range(NBUF)]
        + [pltpu.SemaphoreType.DMA for _ in range(NBUF)]
        + [pltpu.VMEM_SHARED((NP, D), jnp.float32)]
    )

    def body(h_hbm, src_hbm, dst_hbm, *rest):
        if with_deg:
            (p_out, degp_out, dst_v, *rs, acc_sh) = rest
        else:
            (p_out, dst_v, *rs, acc_sh) = rest
        srcs = rs[:NBUF]
        rows = rs[NBUF:2 * NBUF]
        sems = rs[2 * NBUF:]
        cid = lax.axis_index("c")
        sid = lax.axis_index("s")
        wid = sid * NC + cid
        r0 = sid * RPT
        obase = cid * NP

        def zero_acc():
            def zstep(i, _):
                pltpu.sync_copy(rows[0], acc_sh.at[pl.ds(r0 + i * CH, CH)])
                return 0

            lax.fori_loop(0, RCH, zstep, 0)

        def dump_acc(out):
            def dstep(i, _):
                off = r0 + i * CH
                pltpu.sync_copy(acc_sh.at[pl.ds(off, CH)], rows[0])
                pltpu.sync_copy(rows[0], out.at[pl.ds(obase + off, CH)])
                return 0

            lax.fori_loop(0, RCH, dstep, 0)

        pltpu.sync_copy(dst_hbm.at[wid], dst_v)

        _fill_const_2d(rows[0], CH, D, 0.0)
        zero_acc()
        plsc.subcore_barrier()

        def estep(k, _):
            pltpu.sync_copy(src_hbm.at[wid, k], srcs[0])
            pltpu.async_copy(h_hbm.at[srcs[0]], rows[0], sems[0]).wait()
            pltpu.sync_copy(rows[0], acc_sh.at[dst_v.at[k]], add=True)
            return 0

        lax.fori_loop(0, NCHUNK, estep, 0)
        plsc.subcore_barrier()

        dump_acc(p_out)

        if with_deg:
            _fill_const_2d(rows[0], CH, D, 0.0)
            zero_acc()
            plsc.subcore_barrier()
            _fill_const_2d(rows[1], CH, D, 1.0)

            def gstep(k, _):
                pltpu.sync_copy(rows[1], acc_sh.at[dst_v.at[k]], add=True)
                return 0

            lax.fori_loop(0, NCHUNK, gstep, 0)
            plsc.subcore_barrier()
            dump_acc(degp_out)

    return pl.kernel(body, out_type=out_type, mesh=mesh,
                     scratch_types=scratch)


_R = 2000


def _proj_kernel(x_ref, w_ref, o_ref):
    o_ref[...] = jnp.maximum(
        jnp.dot(x_ref[...], w_ref[...], preferred_element_type=jnp.float32),
        0.0)


def _proj_relu(x, w):
    return pl.pallas_call(
        _proj_kernel,
        grid=(N // _R,),
        in_specs=[pl.BlockSpec((_R, D), lambda i: (i, 0)),
                  pl.BlockSpec((D, D), lambda i: (0, 0))],
        out_specs=pl.BlockSpec((_R, D), lambda i: (i, 0)),
        out_shape=jax.ShapeDtypeStruct((N, D), jnp.float32),
    )(x, w)


def _mid_kernel(coeff_ref, p_ref, degp_ref, basis_ref, h_ref, inv_ref):
    w = coeff_ref[0, 0] * basis_ref[0] + coeff_ref[0, 1] * basis_ref[1]
    agg = p_ref[0] + p_ref[1]
    deg = degp_ref[0] + degp_ref[1]
    inv = 1.0 / jnp.maximum(deg, 1.0)
    inv_ref[...] = inv
    m = jnp.dot(agg, w, preferred_element_type=jnp.float32) * inv
    h_ref[...] = jnp.maximum(m, 0.0)


def _mid(p, degp, basis, coeff_row):
    return pl.pallas_call(
        _mid_kernel,
        grid=(N // _R,),
        in_specs=[pl.BlockSpec(memory_space=pltpu.SMEM),
                  pl.BlockSpec((NC, _R, D), lambda i: (0, i, 0)),
                  pl.BlockSpec((NC, _R, D), lambda i: (0, i, 0)),
                  pl.BlockSpec((2, D, D), lambda i: (0, 0, 0))],
        out_specs=[pl.BlockSpec((_R, D), lambda i: (i, 0)),
                   pl.BlockSpec((_R, D), lambda i: (i, 0))],
        out_shape=[jax.ShapeDtypeStruct((N, D), jnp.float32),
                   jax.ShapeDtypeStruct((N, D), jnp.float32)],
    )(coeff_row, p, degp, basis)


def _final_kernel(coeff_ref, p_ref, inv_ref, basis_ref, wdec_ref, bdec_ref,
                  feat_ref, h_ref, loss_ref):
    w = coeff_ref[0, 0] * basis_ref[0] + coeff_ref[0, 1] * basis_ref[1]
    agg = p_ref[0] + p_ref[1]
    h = jnp.dot(agg, w, preferred_element_type=jnp.float32) * inv_ref[...]
    h_ref[...] = h
    rec = (jnp.dot(h, wdec_ref[...], preferred_element_type=jnp.float32)
           + bdec_ref[...])
    dlt = rec - feat_ref[...]
    part = jnp.sum(dlt * dlt) / (N * D)

    @pl.when(pl.program_id(0) == 0)
    def _():
        loss_ref[0, 0] = 0.0

    loss_ref[0, 0] += part


def _final(p, inv, basis, coeff_row, wdec, bdec, feat):
    return pl.pallas_call(
        _final_kernel,
        grid=(N // _R,),
        in_specs=[pl.BlockSpec(memory_space=pltpu.SMEM),
                  pl.BlockSpec((NC, _R, D), lambda i: (0, i, 0)),
                  pl.BlockSpec((_R, D), lambda i: (i, 0)),
                  pl.BlockSpec((2, D, D), lambda i: (0, 0, 0)),
                  pl.BlockSpec((D, D), lambda i: (0, 0)),
                  pl.BlockSpec((1, D), lambda i: (0, 0)),
                  pl.BlockSpec((_R, D), lambda i: (i, 0))],
        out_specs=[pl.BlockSpec((_R, D), lambda i: (i, 0)),
                   pl.BlockSpec(memory_space=pltpu.SMEM)],
        out_shape=[jax.ShapeDtypeStruct((N, D), jnp.float32),
                   jax.ShapeDtypeStruct((1, 1), jnp.float32)],
    )(coeff_row, p, inv, basis, wdec, bdec.reshape(1, D), feat)


def kernel(feat_user, feat_item, edge_u2i, edge_i2u,
           W_in_user, W_in_item, basis1, coeff1, basis2, coeff2,
           W_dec_user, b_dec_user, W_dec_item, b_dec_item):
    def pad_edges(idx, fill):
        idx = idx.astype(jnp.int32)
        pad = jnp.full((NW * EP - E,), fill, jnp.int32)
        return jnp.concatenate([idx, pad]).reshape(NW, NCHUNK, CH)

    su = pad_edges(edge_u2i[0], 0)
    du = pad_edges(edge_u2i[1], N)
    si = pad_edges(edge_i2u[0], 0)
    di = pad_edges(edge_i2u[1], N)

    h_u = _proj_relu(feat_user, W_in_user)
    h_i = _proj_relu(feat_item, W_in_item)

    _sc_conv_deg = _make_sc_conv(True)
    _sc_conv = _make_sc_conv(False)

    p_item, degp_item = _sc_conv_deg(h_u, su, du)
    p_user, degp_user = _sc_conv_deg(h_i, si, di)
    p_item = p_item.reshape(NC, NP, D)
    p_user = p_user.reshape(NC, NP, D)
    degp_item = degp_item.reshape(NC, NP, D)
    degp_user = degp_user.reshape(NC, NP, D)
    h_i1, inv_i = _mid(p_item, degp_item, basis1, coeff1[0:1])
    h_u1, inv_u = _mid(p_user, degp_user, basis1, coeff1[1:2])

    (p_item2,) = _sc_conv(h_u1, su, du)
    (p_user2,) = _sc_conv(h_i1, si, di)
    p_item2 = p_item2.reshape(NC, NP, D)
    p_user2 = p_user2.reshape(NC, NP, D)
    h_i2, loss_i = _final(p_item2, inv_i, basis2, coeff2[0:1],
                          W_dec_item, b_dec_item, feat_item)
    h_u2, loss_u = _final(p_user2, inv_u, basis2, coeff2[1:2],
                          W_dec_user, b_dec_user, feat_user)

    loss = loss_u[0, 0] + loss_i[0, 0]
    return (loss, h_u2, h_i2)


# speedup vs baseline: 3.1093x; 1.5942x over previous
import functools

import jax
import jax.numpy as jnp
from jax import lax
from jax.experimental import pallas as pl
from jax.experimental.pallas import tpu as pltpu
from jax.experimental.pallas import tpu_sc as plsc

N = 10000
D = 128
E = 320000
NC = 2
NS = 16
NW = NC * NS
CH = 80
NCHUNK = 125
EP = NCHUNK * CH
NBUF = 2
NP = 10240
RPT = NP // NS
RCH = RPT // CH


def _fill_const_2d(ref, nrows, ncols, value):
    vec = jnp.full((16,), value, dtype=jnp.float32)

    def row(r, _):
        def col(c, _):
            ref[r, pl.ds(c * 16, 16)] = vec
            return 0
        return lax.fori_loop(0, ncols // 16, col, 0)

    lax.fori_loop(0, nrows, row, 0)


@functools.lru_cache(maxsize=None)
def _make_sc_conv(with_deg):
    mesh = plsc.VectorSubcoreMesh(core_axis_name="c", subcore_axis_name="s",
                                  num_cores=NC, num_subcores=NS)
    out_type = [jax.ShapeDtypeStruct((NC * NP, D), jnp.float32)]
    if with_deg:
        out_type.append(jax.ShapeDtypeStruct((NC * NP, D), jnp.float32))
    scratch = (
        [pltpu.VMEM((NCHUNK, CH), jnp.int32)]
        + [pltpu.VMEM((CH,), jnp.int32) for _ in range(NBUF)]
        + [pltpu.VMEM((CH, D), jnp.float32) for _ in range(NBUF)]
        + [pltpu.SemaphoreType.DMA for _ in range(NBUF)]
        + [pltpu.VMEM_SHARED((NP, D), jnp.float32)]
    )

    def body(h_hbm, src_hbm, dst_hbm, *rest):
        if with_deg:
            (p_out, degp_out, dst_v, *rs, acc_sh) = rest
        else:
            (p_out, dst_v, *rs, acc_sh) = rest
        srcs = rs[:NBUF]
        rows = rs[NBUF:2 * NBUF]
        sems = rs[2 * NBUF:]
        cid = lax.axis_index("c")
        sid = lax.axis_index("s")
        wid = sid * NC + cid
        r0 = sid * RPT
        obase = cid * NP

        def zero_acc():
            def zstep(i, _):
                pltpu.sync_copy(rows[0], acc_sh.at[pl.ds(r0 + i * CH, CH)])
                return 0

            lax.fori_loop(0, RCH, zstep, 0)

        def dump_acc(out):
            def dstep(i, _):
                off = r0 + i * CH
                pltpu.sync_copy(acc_sh.at[pl.ds(off, CH)], rows[0])
                pltpu.sync_copy(rows[0], out.at[pl.ds(obase + off, CH)])
                return 0

            lax.fori_loop(0, RCH, dstep, 0)

        pltpu.sync_copy(dst_hbm.at[wid], dst_v)

        _fill_const_2d(rows[0], CH, D, 0.0)
        zero_acc()
        plsc.subcore_barrier()

        for b in range(NBUF):
            pltpu.sync_copy(src_hbm.at[wid, b], srcs[b])
            pltpu.async_copy(h_hbm.at[srcs[b]], rows[b], sems[b])

        def estep(j, _):
            for b in range(NBUF):
                k = j * NBUF + b
                pltpu.make_async_copy(h_hbm.at[srcs[b]], rows[b],
                                      sems[b]).wait()
                pltpu.sync_copy(rows[b], acc_sh.at[dst_v.at[k]], add=True)

                @pl.when(k + NBUF < NCHUNK)
                def _():
                    pltpu.sync_copy(src_hbm.at[wid, k + NBUF], srcs[b])
                    pltpu.async_copy(h_hbm.at[srcs[b]], rows[b], sems[b])
            return 0

        lax.fori_loop(0, NCHUNK // NBUF, estep, 0)
        for b in range(NCHUNK % NBUF):
            k = (NCHUNK // NBUF) * NBUF + b
            pltpu.make_async_copy(h_hbm.at[srcs[b]], rows[b], sems[b]).wait()
            pltpu.sync_copy(rows[b], acc_sh.at[dst_v.at[k]], add=True)
        plsc.subcore_barrier()

        dump_acc(p_out)

        if with_deg:
            _fill_const_2d(rows[0], CH, D, 0.0)
            zero_acc()
            plsc.subcore_barrier()
            _fill_const_2d(rows[1], CH, D, 1.0)

            def gstep(k, _):
                pltpu.sync_copy(rows[1], acc_sh.at[dst_v.at[k]], add=True)
                return 0

            lax.fori_loop(0, NCHUNK, gstep, 0)
            plsc.subcore_barrier()
            dump_acc(degp_out)

    return pl.kernel(body, out_type=out_type, mesh=mesh,
                     scratch_types=scratch)


_R = 2000


def _proj_kernel(x_ref, w_ref, o_ref):
    o_ref[...] = jnp.maximum(
        jnp.dot(x_ref[...], w_ref[...], preferred_element_type=jnp.float32),
        0.0)


def _proj_relu(x, w):
    return pl.pallas_call(
        _proj_kernel,
        grid=(N // _R,),
        in_specs=[pl.BlockSpec((_R, D), lambda i: (i, 0)),
                  pl.BlockSpec((D, D), lambda i: (0, 0))],
        out_specs=pl.BlockSpec((_R, D), lambda i: (i, 0)),
        out_shape=jax.ShapeDtypeStruct((N, D), jnp.float32),
    )(x, w)


def _mid_kernel(coeff_ref, p_ref, degp_ref, basis_ref, h_ref, inv_ref):
    w = coeff_ref[0, 0] * basis_ref[0] + coeff_ref[0, 1] * basis_ref[1]
    agg = p_ref[0] + p_ref[1]
    deg = degp_ref[0] + degp_ref[1]
    inv = 1.0 / jnp.maximum(deg, 1.0)
    inv_ref[...] = inv
    m = jnp.dot(agg, w, preferred_element_type=jnp.float32) * inv
    h_ref[...] = jnp.maximum(m, 0.0)


def _mid(p, degp, basis, coeff_row):
    return pl.pallas_call(
        _mid_kernel,
        grid=(N // _R,),
        in_specs=[pl.BlockSpec(memory_space=pltpu.SMEM),
                  pl.BlockSpec((NC, _R, D), lambda i: (0, i, 0)),
                  pl.BlockSpec((NC, _R, D), lambda i: (0, i, 0)),
                  pl.BlockSpec((2, D, D), lambda i: (0, 0, 0))],
        out_specs=[pl.BlockSpec((_R, D), lambda i: (i, 0)),
                   pl.BlockSpec((_R, D), lambda i: (i, 0))],
        out_shape=[jax.ShapeDtypeStruct((N, D), jnp.float32),
                   jax.ShapeDtypeStruct((N, D), jnp.float32)],
    )(coeff_row, p, degp, basis)


def _final_kernel(coeff_ref, p_ref, inv_ref, basis_ref, wdec_ref, bdec_ref,
                  feat_ref, h_ref, loss_ref):
    w = coeff_ref[0, 0] * basis_ref[0] + coeff_ref[0, 1] * basis_ref[1]
    agg = p_ref[0] + p_ref[1]
    h = jnp.dot(agg, w, preferred_element_type=jnp.float32) * inv_ref[...]
    h_ref[...] = h
    rec = (jnp.dot(h, wdec_ref[...], preferred_element_type=jnp.float32)
           + bdec_ref[...])
    dlt = rec - feat_ref[...]
    part = jnp.sum(dlt * dlt) / (N * D)

    @pl.when(pl.program_id(0) == 0)
    def _():
        loss_ref[0, 0] = 0.0

    loss_ref[0, 0] += part


def _final(p, inv, basis, coeff_row, wdec, bdec, feat):
    return pl.pallas_call(
        _final_kernel,
        grid=(N // _R,),
        in_specs=[pl.BlockSpec(memory_space=pltpu.SMEM),
                  pl.BlockSpec((NC, _R, D), lambda i: (0, i, 0)),
                  pl.BlockSpec((_R, D), lambda i: (i, 0)),
                  pl.BlockSpec((2, D, D), lambda i: (0, 0, 0)),
                  pl.BlockSpec((D, D), lambda i: (0, 0)),
                  pl.BlockSpec((1, D), lambda i: (0, 0)),
                  pl.BlockSpec((_R, D), lambda i: (i, 0))],
        out_specs=[pl.BlockSpec((_R, D), lambda i: (i, 0)),
                   pl.BlockSpec(memory_space=pltpu.SMEM)],
        out_shape=[jax.ShapeDtypeStruct((N, D), jnp.float32),
                   jax.ShapeDtypeStruct((1, 1), jnp.float32)],
    )(coeff_row, p, inv, basis, wdec, bdec.reshape(1, D), feat)


def kernel(feat_user, feat_item, edge_u2i, edge_i2u,
           W_in_user, W_in_item, basis1, coeff1, basis2, coeff2,
           W_dec_user, b_dec_user, W_dec_item, b_dec_item):
    def pad_edges(idx, fill):
        idx = idx.astype(jnp.int32)
        pad = jnp.full((NW * EP - E,), fill, jnp.int32)
        return jnp.concatenate([idx, pad]).reshape(NW, NCHUNK, CH)

    su = pad_edges(edge_u2i[0], 0)
    du = pad_edges(edge_u2i[1], N)
    si = pad_edges(edge_i2u[0], 0)
    di = pad_edges(edge_i2u[1], N)

    h_u = _proj_relu(feat_user, W_in_user)
    h_i = _proj_relu(feat_item, W_in_item)

    _sc_conv_deg = _make_sc_conv(True)
    _sc_conv = _make_sc_conv(False)

    p_item, degp_item = _sc_conv_deg(h_u, su, du)
    p_user, degp_user = _sc_conv_deg(h_i, si, di)
    p_item = p_item.reshape(NC, NP, D)
    p_user = p_user.reshape(NC, NP, D)
    degp_item = degp_item.reshape(NC, NP, D)
    degp_user = degp_user.reshape(NC, NP, D)
    h_i1, inv_i = _mid(p_item, degp_item, basis1, coeff1[0:1])
    h_u1, inv_u = _mid(p_user, degp_user, basis1, coeff1[1:2])

    (p_item2,) = _sc_conv(h_u1, su, du)
    (p_user2,) = _sc_conv(h_i1, si, di)
    p_item2 = p_item2.reshape(NC, NP, D)
    p_user2 = p_user2.reshape(NC, NP, D)
    h_i2, loss_i = _final(p_item2, inv_i, basis2, coeff2[0:1],
                          W_dec_item, b_dec_item, feat_item)
    h_u2, loss_u = _final(p_user2, inv_u, basis2, coeff2[1:2],
                          W_dec_user, b_dec_user, feat_user)

    loss = loss_u[0, 0] + loss_i[0, 0]
    return (loss, h_u2, h_i2)
